# baseline (speedup 1.0000x reference)
import functools

import jax
import jax.numpy as jnp
from jax import lax
from jax.experimental import pallas as pl
from jax.experimental.pallas import tpu as pltpu

NUM_BINS = 255
MIN_V = -20.0
MAX_V = 20.0
BIN_WIDTH = (MAX_V - MIN_V) / (NUM_BINS - 1)

N = 262144
BR = 1024


def _tc_body(vals_ref, out_ref):
    v = vals_ref[:]
    v = jnp.minimum(jnp.maximum(v, MIN_V), MAX_V)
    norm = (v - MIN_V) / BIN_WIDTH
    norm2 = lax.broadcast_in_dim(norm, (BR, NUM_BINS), (0,))
    colf = lax.broadcasted_iota(jnp.int32, (BR, NUM_BINS), 1).astype(jnp.float32)
    out_ref[...] = jnp.maximum(1.0 - jnp.abs(norm2 - colf), 0.0)


_tc_call = pl.pallas_call(
    _tc_body,
    out_shape=jax.ShapeDtypeStruct((N, NUM_BINS), jnp.float32),
    grid=(N // BR,),
    in_specs=[pl.BlockSpec((BR,), lambda i: (i,))],
    out_specs=pl.BlockSpec((BR, NUM_BINS), lambda i: (i, 0)),
)


def kernel(values):
    return _tc_call(values)
